# speedup vs baseline: 1.0209x; 1.0209x over previous
import functools

import jax
import jax.numpy as jnp
from jax import lax
from jax.experimental import pallas as pl
from jax.experimental.pallas import tpu as pltpu
from jax.experimental.pallas import tpu_sc as plsc

D = 2048
S = 2048
NC = 2
NS = 16
NW = NC * NS
CH = 16
LANES = 16


def _sc_body(ids_hbm, wte_hbm, wpe_hbm, out_hbm, idx_v, rows_v, wpe_v, sem):
    n_rows = ids_hbm.shape[0]
    per_w = n_rows // NW
    wid = lax.axis_index("s") * NC + lax.axis_index("c")
    base = wid * per_w
    pos_base = lax.rem(base, S)

    pltpu.sync_copy(ids_hbm.at[pl.ds(base, per_w)], idx_v)

    def chunk(c, _):
        off = c * CH
        gat = pltpu.async_copy(
            wte_hbm.at[idx_v.at[pl.ds(off, CH)]], rows_v, sem)
        pltpu.sync_copy(wpe_hbm.at[pl.ds(pos_base + off, CH)], wpe_v)
        gat.wait()

        @plsc.parallel_loop(0, CH)
        def row(r):
            @plsc.parallel_loop(0, D, LANES, unroll=8)
            def col(j):
                rows_v[r, pl.ds(j, LANES)] = (
                    rows_v[r, pl.ds(j, LANES)] + wpe_v[r, pl.ds(j, LANES)])

        pltpu.sync_copy(rows_v, out_hbm.at[pl.ds(base + off, CH)])
        return 0

    lax.fori_loop(0, per_w // CH, chunk, 0)


def kernel(input_ids, wte, wpe):
    b, s = input_ids.shape
    ids_flat = input_ids.reshape(-1).astype(jnp.int32)
    n_rows = b * s

    mesh = plsc.VectorSubcoreMesh(core_axis_name="c", subcore_axis_name="s")
    out = pl.kernel(
        _sc_body,
        out_type=jax.ShapeDtypeStruct((n_rows, D), jnp.float32),
        mesh=mesh,
        scratch_types=[
            pltpu.VMEM((n_rows // NW,), jnp.int32),
            pltpu.VMEM((CH, D), jnp.float32),
            pltpu.VMEM((CH, D), jnp.float32),
            pltpu.SemaphoreType.DMA,
        ],
    )(ids_flat, wte, wpe)
    return out.reshape(b, s, D)


# speedup vs baseline: 1.1179x; 1.0951x over previous
import functools

import jax
import jax.numpy as jnp
from jax import lax
from jax.experimental import pallas as pl
from jax.experimental.pallas import tpu as pltpu
from jax.experimental.pallas import tpu_sc as plsc

D = 2048
S = 2048
B = 4
NC = 2
NS = 16
NW = NC * NS
CH = 16
LANES = 16
PER_W = S // NW


def _sc_body(ids_hbm, wte_hbm, wpe_hbm, out_hbm, idx_v, rows_v, wpe_v, sem):
    wid = lax.axis_index("s") * NC + lax.axis_index("c")
    pos0 = wid * PER_W

    for b in range(B):
        pltpu.sync_copy(ids_hbm.at[pl.ds(b * S + pos0, PER_W)],
                        idx_v.at[pl.ds(b * PER_W, PER_W)])

    def chunk(c, _):
        off = c * CH
        pltpu.sync_copy(wpe_hbm.at[pl.ds(pos0 + off, CH)], wpe_v)
        for b in range(B):
            pltpu.async_copy(
                wte_hbm.at[idx_v.at[pl.ds(b * PER_W + off, CH)]],
                rows_v, sem).wait()

            @plsc.parallel_loop(0, CH)
            def row(r):
                @plsc.parallel_loop(0, D, LANES, unroll=8)
                def col(j):
                    rows_v[r, pl.ds(j, LANES)] = (
                        rows_v[r, pl.ds(j, LANES)] + wpe_v[r, pl.ds(j, LANES)])

            pltpu.sync_copy(rows_v, out_hbm.at[pl.ds(b * S + pos0 + off, CH)])
        return 0

    lax.fori_loop(0, PER_W // CH, chunk, 0)


def kernel(input_ids, wte, wpe):
    b, s = input_ids.shape
    ids_flat = input_ids.reshape(-1).astype(jnp.int32)
    n_rows = b * s

    mesh = plsc.VectorSubcoreMesh(core_axis_name="c", subcore_axis_name="s")
    out = pl.kernel(
        _sc_body,
        out_type=jax.ShapeDtypeStruct((n_rows, D), jnp.float32),
        mesh=mesh,
        scratch_types=[
            pltpu.VMEM((B * PER_W,), jnp.int32),
            pltpu.VMEM((CH, D), jnp.float32),
            pltpu.VMEM((CH, D), jnp.float32),
            pltpu.SemaphoreType.DMA,
        ],
    )(ids_flat, wte, wpe)
    return out.reshape(b, s, D)


# speedup vs baseline: 1.3908x; 1.2441x over previous
import functools

import jax
import jax.numpy as jnp
from jax import lax
from jax.experimental import pallas as pl
from jax.experimental.pallas import tpu as pltpu
from jax.experimental.pallas import tpu_sc as plsc

D = 2048
S = 2048
B = 4
NC = 2
NS = 16
NW = NC * NS
CH = 16
LANES = 16
PER_W = S // NW


def _sc_body(ids_hbm, wte_hbm, wpe_hbm, out_hbm, idx_v, rows0, rows1, wpe_v,
             gsem0, gsem1, ssem0, ssem1):
    wid = lax.axis_index("s") * NC + lax.axis_index("c")
    pos0 = wid * PER_W

    for b in range(B):
        pltpu.sync_copy(ids_hbm.at[pl.ds(b * S + pos0, PER_W)],
                        idx_v.at[pl.ds(b * PER_W, PER_W)])

    rows = (rows0, rows1)
    gsem = (gsem0, gsem1)
    ssem = (ssem0, ssem1)
    n_steps = (PER_W // CH) * B

    def idx_slice(i):
        c, b = divmod(i, B)
        return idx_v.at[pl.ds(b * PER_W + c * CH, CH)]

    def out_slice(i):
        c, b = divmod(i, B)
        return out_hbm.at[pl.ds(b * S + pos0 + c * CH, CH)]

    gat = [None, None]
    sto = [None, None]
    gat[0] = pltpu.async_copy(wte_hbm.at[idx_slice(0)], rows[0], gsem[0])
    for i in range(n_steps):
        p, q = i % 2, (i + 1) % 2
        if sto[q] is not None:
            sto[q].wait()
        if i + 1 < n_steps:
            gat[q] = pltpu.async_copy(wte_hbm.at[idx_slice(i + 1)], rows[q],
                                      gsem[q])
        gat[p].wait()
        if i % B == 0:
            pltpu.sync_copy(wpe_hbm.at[pl.ds(pos0 + (i // B) * CH, CH)], wpe_v)
        buf = rows[p]

        @plsc.parallel_loop(0, CH)
        def row(r):
            @plsc.parallel_loop(0, D, LANES, unroll=8)
            def col(j):
                buf[r, pl.ds(j, LANES)] = (
                    buf[r, pl.ds(j, LANES)] + wpe_v[r, pl.ds(j, LANES)])

        sto[p] = pltpu.async_copy(buf, out_slice(i), ssem[p])
    sto[(n_steps - 1) % 2].wait()


def kernel(input_ids, wte, wpe):
    b, s = input_ids.shape
    ids_flat = input_ids.reshape(-1).astype(jnp.int32)
    n_rows = b * s

    mesh = plsc.VectorSubcoreMesh(core_axis_name="c", subcore_axis_name="s")
    out = pl.kernel(
        _sc_body,
        out_type=jax.ShapeDtypeStruct((n_rows, D), jnp.float32),
        mesh=mesh,
        scratch_types=[
            pltpu.VMEM((B * PER_W,), jnp.int32),
            pltpu.VMEM((CH, D), jnp.float32),
            pltpu.VMEM((CH, D), jnp.float32),
            pltpu.VMEM((CH, D), jnp.float32),
            pltpu.SemaphoreType.DMA,
            pltpu.SemaphoreType.DMA,
            pltpu.SemaphoreType.DMA,
            pltpu.SemaphoreType.DMA,
        ],
    )(ids_flat, wte, wpe)
    return out.reshape(b, s, D)
